# baseline (speedup 1.0000x reference)
import functools

import jax
import jax.numpy as jnp
from jax.experimental import pallas as pl
from jax.experimental.pallas import tpu as pltpu


def _sigmoid(x):
    return 0.5 * jnp.tanh(0.5 * x) + 0.5


def _fused_kernel(len_ref, xr_ref, xw_ref, WrT_ref, brb_ref, W2e_ref,
                  W2o_ref, WhhT_ref, bg_ref, WcT_ref, bc_ref, gamma_ref,
                  beta_ref, out_ref, *, T, H):
    br = jax.nn.relu(
        jnp.dot(xr_ref[...], WrT_ref[...],
                preferred_element_type=jnp.float32) + brb_ref[...])

    lens = len_ref[...]
    x = xw_ref[...]
    B = x.shape[0]
    P = x.shape[1]

    xt = jnp.transpose(x, (1, 0, 2)).reshape(P * B, -1)
    W2e = W2e_ref[...]
    W2o = W2o_ref[...]
    bg = bg_ref[...]

    h = jnp.zeros((B, H), dtype=jnp.float32)
    c = jnp.zeros((B, H), dtype=jnp.float32)
    hF = jnp.zeros((B, H), dtype=jnp.float32)
    WhhT = WhhT_ref[...]

    for p in range(P):
        rows = xt[p * B:(p + 1) * B, :]
        for sub, Wsub in ((0, W2e), (1, W2o)):
            t = 2 * p + sub
            gates = (jnp.dot(rows, Wsub,
                             preferred_element_type=jnp.float32)
                     + jnp.dot(h, WhhT, preferred_element_type=jnp.float32)
                     + bg)
            tg = jnp.tanh(gates)
            s_i = 0.5 * tg[:, 0 * H:1 * H] + 0.5
            s_f = 0.5 * tg[:, 1 * H:2 * H] + 0.5
            t_g = tg[:, 2 * H:3 * H]
            s_o = 0.5 * tg[:, 3 * H:4 * H] + 0.5
            c = s_f * c + s_i * t_g
            h = s_o * jnp.tanh(c)
            hF = jnp.where(lens == t + 1, h, hF)

    hb = jnp.concatenate([br, hF], axis=1)
    out = jnp.dot(hb, WcT_ref[...],
                  preferred_element_type=jnp.float32) + bc_ref[...]
    mu = jnp.mean(out, axis=1, keepdims=True)
    d = out - mu
    var = jnp.mean(d * d, axis=1, keepdims=True)
    y = d * jax.lax.rsqrt(var + 1e-5) * gamma_ref[...] + beta_ref[...]
    out_ref[...] = y


def kernel(xr, xw, xn, Wr, br_b, W_ih, W_hh, b_ih, b_hh, Wc, bc, gamma, beta):
    BS, SL, DR = xr.shape
    T, DV = xw.shape[2], xw.shape[3]
    H = Wr.shape[0]
    DH = Wc.shape[0]
    N = BS * SL
    B = 1024
    nblocks = N // B

    xr2 = xr.reshape(N, DR)
    xw2 = xw.reshape(N, T // 2, 2 * DV)
    lens2 = jnp.broadcast_to(
        xn[:, :, -1].reshape(N, 1).astype(jnp.int32), (N, H))

    WrT = Wr.T
    gate_scale = jnp.concatenate([
        jnp.full((H,), 0.5), jnp.full((H,), 0.5),
        jnp.ones((H,)), jnp.full((H,), 0.5)]).astype(jnp.float32)
    WihT = (W_ih * gate_scale[:, None]).T
    Wz = jnp.zeros_like(WihT)
    W2e = jnp.concatenate([WihT, Wz], axis=0)
    W2o = jnp.concatenate([Wz, WihT], axis=0)
    WhhT = (W_hh * gate_scale[:, None]).T
    bg = ((b_ih + b_hh) * gate_scale).reshape(1, 4 * H)
    WcT = Wc.T

    out = pl.pallas_call(
        functools.partial(_fused_kernel, T=T, H=H),
        grid=(nblocks,),
        in_specs=[
            pl.BlockSpec((B, H), lambda i: (i, 0)),
            pl.BlockSpec((B, DR), lambda i: (i, 0)),
            pl.BlockSpec((B, T // 2, 2 * DV), lambda i: (i, 0, 0)),
            pl.BlockSpec((DR, H), lambda i: (0, 0)),
            pl.BlockSpec((1, H), lambda i: (0, 0)),
            pl.BlockSpec((2 * DV, 4 * H), lambda i: (0, 0)),
            pl.BlockSpec((2 * DV, 4 * H), lambda i: (0, 0)),
            pl.BlockSpec((H, 4 * H), lambda i: (0, 0)),
            pl.BlockSpec((1, 4 * H), lambda i: (0, 0)),
            pl.BlockSpec((DH, DH), lambda i: (0, 0)),
            pl.BlockSpec((1, DH), lambda i: (0, 0)),
            pl.BlockSpec((1, DH), lambda i: (0, 0)),
            pl.BlockSpec((1, DH), lambda i: (0, 0)),
        ],
        out_specs=pl.BlockSpec((B, DH), lambda i: (i, 0)),
        out_shape=jax.ShapeDtypeStruct((N, DH), jnp.float32),
        compiler_params=pltpu.CompilerParams(
            dimension_semantics=("parallel",)),
    )(lens2, xr2, xw2, WrT, br_b.reshape(1, H), W2e, W2o, WhhT, bg, WcT,
      bc.reshape(1, DH), gamma.reshape(1, DH), beta.reshape(1, DH))

    return out.reshape(BS, SL, DH)


# speedup vs baseline: 1.4312x; 1.4312x over previous
import functools

import jax
import jax.numpy as jnp
from jax.experimental import pallas as pl
from jax.experimental.pallas import tpu as pltpu


def _sigmoid(x):
    return 0.5 * jnp.tanh(0.5 * x) + 0.5


def _fused_kernel(len_ref, xr_ref, xw_ref, WrT_ref, brb_ref, WihT_ref,
                  WhhT_ref, bg_ref, WcT_ref, bc_ref, gamma_ref, beta_ref,
                  out_ref, *, T, H):
    br = jax.nn.relu(
        jnp.dot(xr_ref[...], WrT_ref[...],
                preferred_element_type=jnp.float32) + brb_ref[...])

    lens = len_ref[...]
    x = xw_ref[...]
    B = x.shape[0]

    xt = jnp.transpose(x, (1, 0, 2)).reshape(T * B, -1)
    WihT = WihT_ref[...]
    bg = bg_ref[...]

    h = jnp.zeros((B, H), dtype=jnp.float32)
    c = jnp.zeros((B, H), dtype=jnp.float32)
    hF = jnp.zeros((B, H), dtype=jnp.float32)
    WhhT = WhhT_ref[...]

    for t in range(T):
        gates = (jnp.dot(xt[t * B:(t + 1) * B, :], WihT,
                         preferred_element_type=jnp.float32)
                 + jnp.dot(h, WhhT, preferred_element_type=jnp.float32)
                 + bg)
        tg = jnp.tanh(gates)
        s_i = 0.5 * tg[:, 0 * H:1 * H] + 0.5
        s_f = 0.5 * tg[:, 1 * H:2 * H] + 0.5
        t_g = tg[:, 2 * H:3 * H]
        s_o = 0.5 * tg[:, 3 * H:4 * H] + 0.5
        c = s_f * c + s_i * t_g
        h = s_o * jnp.tanh(c)
        hF = jnp.where(lens == t + 1, h, hF)

    hb = jnp.concatenate([br, hF], axis=1)
    out = jnp.dot(hb, WcT_ref[...],
                  preferred_element_type=jnp.float32) + bc_ref[...]
    mu = jnp.mean(out, axis=1, keepdims=True)
    d = out - mu
    var = jnp.mean(d * d, axis=1, keepdims=True)
    y = d * jax.lax.rsqrt(var + 1e-5) * gamma_ref[...] + beta_ref[...]
    out_ref[...] = y


def kernel(xr, xw, xn, Wr, br_b, W_ih, W_hh, b_ih, b_hh, Wc, bc, gamma, beta):
    BS, SL, DR = xr.shape
    T, DV = xw.shape[2], xw.shape[3]
    H = Wr.shape[0]
    DH = Wc.shape[0]
    N = BS * SL
    B = 1024
    nblocks = N // B

    xr2 = xr.reshape(N, DR)
    xw2 = xw.reshape(N, T, DV)
    lens2 = jnp.broadcast_to(
        xn[:, :, -1].reshape(N, 1).astype(jnp.int32), (N, H))

    WrT = Wr.T
    gate_scale = jnp.concatenate([
        jnp.full((H,), 0.5), jnp.full((H,), 0.5),
        jnp.ones((H,)), jnp.full((H,), 0.5)]).astype(jnp.float32)
    WihT = (W_ih * gate_scale[:, None]).T
    WhhT = (W_hh * gate_scale[:, None]).T
    bg = ((b_ih + b_hh) * gate_scale).reshape(1, 4 * H)
    WcT = Wc.T

    out = pl.pallas_call(
        functools.partial(_fused_kernel, T=T, H=H),
        grid=(nblocks,),
        in_specs=[
            pl.BlockSpec((B, H), lambda i: (i, 0)),
            pl.BlockSpec((B, DR), lambda i: (i, 0)),
            pl.BlockSpec((B, T, DV), lambda i: (i, 0, 0)),
            pl.BlockSpec((DR, H), lambda i: (0, 0)),
            pl.BlockSpec((1, H), lambda i: (0, 0)),
            pl.BlockSpec((DV, 4 * H), lambda i: (0, 0)),
            pl.BlockSpec((H, 4 * H), lambda i: (0, 0)),
            pl.BlockSpec((1, 4 * H), lambda i: (0, 0)),
            pl.BlockSpec((DH, DH), lambda i: (0, 0)),
            pl.BlockSpec((1, DH), lambda i: (0, 0)),
            pl.BlockSpec((1, DH), lambda i: (0, 0)),
            pl.BlockSpec((1, DH), lambda i: (0, 0)),
        ],
        out_specs=pl.BlockSpec((B, DH), lambda i: (i, 0)),
        out_shape=jax.ShapeDtypeStruct((N, DH), jnp.float32),
        compiler_params=pltpu.CompilerParams(
            dimension_semantics=("parallel",)),
    )(lens2, xr2, xw2, WrT, br_b.reshape(1, H), WihT, WhhT, bg, WcT,
      bc.reshape(1, DH), gamma.reshape(1, DH), beta.reshape(1, DH))

    return out.reshape(BS, SL, DH)


# speedup vs baseline: 1.4349x; 1.0026x over previous
import functools

import jax
import jax.numpy as jnp
from jax.experimental import pallas as pl
from jax.experimental.pallas import tpu as pltpu


def _fused_kernel(len_ref, xr_ref, xw_ref, WrT_ref, brb_ref, WihT_ref,
                  WhhT_ref, bg_ref, WcT_ref, bc_ref, gamma_ref, beta_ref,
                  out_ref, *, T, H):
    br = jax.nn.relu(
        jnp.dot(xr_ref[...], WrT_ref[...],
                preferred_element_type=jnp.float32) + brb_ref[...])

    lens = len_ref[...]
    x = xw_ref[...]
    B = x.shape[0]

    xt = jnp.transpose(x, (1, 0, 2)).reshape(T * B, -1)
    WihT = WihT_ref[...]
    bg = bg_ref[...]

    h = jnp.zeros((B, H), dtype=jnp.float32)
    c = jnp.zeros((B, H), dtype=jnp.float32)
    hF = jnp.zeros((B, H), dtype=jnp.float32)
    WhhT = WhhT_ref[...]

    for t in range(T):
        gates = (jnp.dot(xt[t * B:(t + 1) * B, :], WihT,
                         preferred_element_type=jnp.float32)
                 + jnp.dot(h, WhhT, preferred_element_type=jnp.float32)
                 + bg)
        tg = jnp.tanh(gates)
        s_i = 0.5 * tg[:, 0 * H:1 * H] + 0.5
        s_f = 0.5 * tg[:, 1 * H:2 * H] + 0.5
        t_g = tg[:, 2 * H:3 * H]
        s_o = 0.5 * tg[:, 3 * H:4 * H] + 0.5
        c = s_f * c + s_i * t_g
        h = s_o * jnp.tanh(c)
        hF = jnp.where(lens == t + 1, h, hF)

    hb = jnp.concatenate([br, hF], axis=1)
    out = jnp.dot(hb, WcT_ref[...],
                  preferred_element_type=jnp.float32) + bc_ref[...]
    mu = jnp.mean(out, axis=1, keepdims=True)
    d = out - mu
    var = jnp.mean(d * d, axis=1, keepdims=True)
    y = d * jax.lax.rsqrt(var + 1e-5) * gamma_ref[...] + beta_ref[...]
    out_ref[...] = y


def kernel(xr, xw, xn, Wr, br_b, W_ih, W_hh, b_ih, b_hh, Wc, bc, gamma, beta):
    BS, SL, DR = xr.shape
    T, DV = xw.shape[2], xw.shape[3]
    H = Wr.shape[0]
    DH = Wc.shape[0]
    N = BS * SL
    B = 1024
    nblocks = N // B

    xr2 = xr.reshape(N, DR)
    xw2 = xw.reshape(N, T, DV)
    lens2 = jnp.broadcast_to(
        xn[:, :, -1].reshape(N, 1).astype(jnp.int32), (N, H))

    WrT = Wr.T
    gate_scale = jnp.concatenate([
        jnp.full((H,), 0.5), jnp.full((H,), 0.5),
        jnp.ones((H,)), jnp.full((H,), 0.5)]).astype(jnp.float32)
    WihT = (W_ih * gate_scale[:, None]).T
    WhhT = (W_hh * gate_scale[:, None]).T
    bg = ((b_ih + b_hh) * gate_scale).reshape(1, 4 * H)
    WcT = Wc.T

    out = pl.pallas_call(
        functools.partial(_fused_kernel, T=T, H=H),
        grid=(nblocks,),
        in_specs=[
            pl.BlockSpec((B, H), lambda i: (i, 0)),
            pl.BlockSpec((B, DR), lambda i: (i, 0)),
            pl.BlockSpec((B, T, DV), lambda i: (i, 0, 0)),
            pl.BlockSpec((DR, H), lambda i: (0, 0)),
            pl.BlockSpec((1, H), lambda i: (0, 0)),
            pl.BlockSpec((DV, 4 * H), lambda i: (0, 0)),
            pl.BlockSpec((H, 4 * H), lambda i: (0, 0)),
            pl.BlockSpec((1, 4 * H), lambda i: (0, 0)),
            pl.BlockSpec((DH, DH), lambda i: (0, 0)),
            pl.BlockSpec((1, DH), lambda i: (0, 0)),
            pl.BlockSpec((1, DH), lambda i: (0, 0)),
            pl.BlockSpec((1, DH), lambda i: (0, 0)),
        ],
        out_specs=pl.BlockSpec((B, DH), lambda i: (i, 0)),
        out_shape=jax.ShapeDtypeStruct((N, DH), jnp.float32),
        compiler_params=pltpu.CompilerParams(
            dimension_semantics=("parallel",)),
    )(lens2, xr2, xw2, WrT, br_b.reshape(1, H), WihT, WhhT, bg, WcT,
      bc.reshape(1, DH), gamma.reshape(1, DH), beta.reshape(1, DH))

    return out.reshape(BS, SL, DH)
